# Initial kernel scaffold; baseline (speedup 1.0000x reference)
import jax
import jax.numpy as jnp
from jax.experimental import pallas as pl


def kernel(z_e, codebook):
    raise NotImplementedError("write your pallas kernel here")


# speedup vs baseline: 3.6502x; 3.6502x over previous
import functools

import jax
import jax.numpy as jnp
from jax import lax
from jax.experimental import pallas as pl
from jax.experimental.pallas import tpu as pltpu
from jax.experimental.pallas import tpu_sc as plsc

_NPTS = 16384
_D = 32
_K = 8192
_M_BLOCK = 512
_N_CHUNK = 2048

_NC = 2
_NS = 16
_NW = _NC * _NS
_PTS_PER_W = _NPTS // _NW
_GATHER_CHUNK = 128
_ROWS_PER_W = _PTS_PER_W // _GATHER_CHUNK


def _argmin_body(zt_ref, cb_ref, out_ref):
    zb = zt_ref[...]
    znorm = jnp.sum(zb * zb, axis=0, keepdims=True)

    def body(j, carry):
        m, bi = carry
        cb = cb_ref[pl.ds(j * _N_CHUNK, _N_CHUNK), :]
        cnorm = jnp.sum(cb * cb, axis=1, keepdims=True)
        s = lax.dot_general(cb, zb, (((1,), (0,)), ((), ())),
                            preferred_element_type=jnp.float32)
        d = (znorm + cnorm) - 2.0 * s
        cm = jnp.min(d, axis=0, keepdims=True)
        jidx = lax.broadcasted_iota(jnp.int32, (_N_CHUNK, _M_BLOCK), 0)
        cidx = jnp.min(jnp.where(d == cm, jidx + j * _N_CHUNK,
                                 jnp.int32(2**30)),
                       axis=0, keepdims=True)
        upd = cm < m
        return jnp.where(upd, cm, m), jnp.where(upd, cidx, bi)

    m0 = jnp.full((1, _M_BLOCK), jnp.inf, dtype=jnp.float32)
    i0 = jnp.zeros((1, _M_BLOCK), dtype=jnp.int32)
    _, bi = lax.fori_loop(0, _K // _N_CHUNK, body, (m0, i0))
    out_ref[...] = bi.reshape(1, 1, _M_BLOCK)


def _encode_indices(z2dt, codebook):
    n_blocks = _NPTS // _M_BLOCK
    out = pl.pallas_call(
        _argmin_body,
        grid=(n_blocks,),
        in_specs=[
            pl.BlockSpec((_D, _M_BLOCK), lambda g: (0, g)),
            pl.BlockSpec((_K, _D), lambda g: (0, 0)),
        ],
        out_specs=pl.BlockSpec((1, 1, _M_BLOCK), lambda g: (g, 0, 0)),
        out_shape=jax.ShapeDtypeStruct((n_blocks, 1, _M_BLOCK), jnp.int32),
    )(z2dt, codebook)
    return out.reshape(_NPTS)


_DPAD = 128


@functools.cache
def _make_gather_kernel():
    mesh = plsc.VectorSubcoreMesh(core_axis_name="c", subcore_axis_name="s")

    @functools.partial(
        pl.kernel,
        mesh=mesh,
        out_type=jax.ShapeDtypeStruct(
            (_NPTS // _GATHER_CHUNK, _GATHER_CHUNK, _DPAD), jnp.float32),
        scratch_types=[
            pltpu.VMEM((_ROWS_PER_W, _GATHER_CHUNK), jnp.int32),
            pltpu.VMEM((_ROWS_PER_W, _GATHER_CHUNK, _DPAD), jnp.float32),
            pltpu.SemaphoreType.DMA,
        ],
    )
    def _gather_kernel(idx_hbm, table_hbm, out_hbm, idx_v, rows_v, sem):
        wid = lax.axis_index("s") * _NC + lax.axis_index("c")
        base = wid * _ROWS_PER_W
        pltpu.sync_copy(idx_hbm.at[pl.ds(base, _ROWS_PER_W)], idx_v)
        for c in range(_ROWS_PER_W):
            pltpu.async_copy(table_hbm.at[idx_v.at[c]], rows_v.at[c],
                             sem).wait()
        pltpu.sync_copy(rows_v, out_hbm.at[pl.ds(base, _ROWS_PER_W)])

    return _gather_kernel


def kernel(z_e, codebook):
    b, c, h, w = z_e.shape
    z2dt = z_e.reshape(b, c, h * w).transpose(1, 0, 2).reshape(c, b * h * w)
    idx = _encode_indices(z2dt, codebook)
    idx2d = idx.reshape(_NPTS // _GATHER_CHUNK, _GATHER_CHUNK)
    cb_pad = jnp.pad(codebook, ((0, 0), (0, _DPAD - _D)))
    quantized = _make_gather_kernel()(idx2d, cb_pad)
    quantized = quantized.reshape(_NPTS, _DPAD)[:, :_D]
    q = quantized.reshape(b, h * w, c).transpose(0, 2, 1).reshape(b, c, h, w)
    return q


# speedup vs baseline: 4.2632x; 1.1679x over previous
import functools

import jax
import jax.numpy as jnp
from jax import lax
from jax.experimental import pallas as pl
from jax.experimental.pallas import tpu as pltpu
from jax.experimental.pallas import tpu_sc as plsc

_NPTS = 16384
_D = 32
_K = 8192
_M_BLOCK = 512
_N_CHUNK = 2048

_NC = 2
_NS = 16
_NW = _NC * _NS
_PTS_PER_W = _NPTS // _NW
_GATHER_CHUNK = 128
_ROWS_PER_W = _PTS_PER_W // _GATHER_CHUNK


def _argmin_body(zt_ref, cb_ref, out_ref, cnorm_ref):
    @pl.when(pl.program_id(0) == 0)
    def _():
        cb_all = cb_ref[...]
        cnorm_ref[...] = jnp.sum(cb_all * cb_all, axis=1, keepdims=True)

    zb = zt_ref[...]
    znorm = jnp.sum(zb * zb, axis=0, keepdims=True)
    zb2 = zb + zb
    jidx = lax.broadcasted_iota(jnp.int32, (_N_CHUNK, _M_BLOCK), 0)

    m = jnp.full((1, _M_BLOCK), jnp.inf, dtype=jnp.float32)
    bi = jnp.zeros((1, _M_BLOCK), dtype=jnp.int32)
    for j in range(_K // _N_CHUNK):
        cb = cb_ref[pl.ds(j * _N_CHUNK, _N_CHUNK), :]
        cnorm = cnorm_ref[pl.ds(j * _N_CHUNK, _N_CHUNK), :]
        s2 = lax.dot_general(cb, zb2, (((1,), (0,)), ((), ())),
                             preferred_element_type=jnp.float32)
        d = (znorm + cnorm) - s2
        cm = jnp.min(d, axis=0, keepdims=True)
        cidx = jnp.min(jnp.where(d == cm, jidx, jnp.int32(2**30)),
                       axis=0, keepdims=True) + j * _N_CHUNK
        upd = cm < m
        m = jnp.where(upd, cm, m)
        bi = jnp.where(upd, cidx, bi)

    out_ref[...] = bi.reshape(1, 1, _M_BLOCK)


def _encode_indices(z2dt, codebook):
    n_blocks = _NPTS // _M_BLOCK
    out = pl.pallas_call(
        _argmin_body,
        grid=(n_blocks,),
        in_specs=[
            pl.BlockSpec((_D, _M_BLOCK), lambda g: (0, g)),
            pl.BlockSpec((_K, _D), lambda g: (0, 0)),
        ],
        out_specs=pl.BlockSpec((1, 1, _M_BLOCK), lambda g: (g, 0, 0)),
        out_shape=jax.ShapeDtypeStruct((n_blocks, 1, _M_BLOCK), jnp.int32),
        scratch_shapes=[pltpu.VMEM((_K, 1), jnp.float32)],
    )(z2dt, codebook)
    return out.reshape(_NPTS)


_DPAD = 128


@functools.cache
def _make_gather_kernel():
    mesh = plsc.VectorSubcoreMesh(core_axis_name="c", subcore_axis_name="s")

    @functools.partial(
        pl.kernel,
        mesh=mesh,
        out_type=jax.ShapeDtypeStruct(
            (_NPTS // _GATHER_CHUNK, _GATHER_CHUNK, _DPAD), jnp.float32),
        scratch_types=[
            pltpu.VMEM((_ROWS_PER_W, _GATHER_CHUNK), jnp.int32),
            pltpu.VMEM((_ROWS_PER_W, _GATHER_CHUNK, _DPAD), jnp.float32),
            pltpu.SemaphoreType.DMA,
        ],
    )
    def _gather_kernel(idx_hbm, table_hbm, out_hbm, idx_v, rows_v, sem):
        wid = lax.axis_index("s") * _NC + lax.axis_index("c")
        base = wid * _ROWS_PER_W
        pltpu.sync_copy(idx_hbm.at[pl.ds(base, _ROWS_PER_W)], idx_v)
        for c in range(_ROWS_PER_W):
            pltpu.async_copy(table_hbm.at[idx_v.at[c]], rows_v.at[c],
                             sem).wait()
        pltpu.sync_copy(rows_v, out_hbm.at[pl.ds(base, _ROWS_PER_W)])

    return _gather_kernel


def kernel(z_e, codebook):
    b, c, h, w = z_e.shape
    z2dt = z_e.reshape(b, c, h * w).transpose(1, 0, 2).reshape(c, b * h * w)
    idx = _encode_indices(z2dt, codebook)
    idx2d = idx.reshape(_NPTS // _GATHER_CHUNK, _GATHER_CHUNK)
    cb_pad = jnp.pad(codebook, ((0, 0), (0, _DPAD - _D)))
    quantized = _make_gather_kernel()(idx2d, cb_pad)
    quantized = quantized.reshape(_NPTS, _DPAD)[:, :_D]
    q = quantized.reshape(b, h * w, c).transpose(0, 2, 1).reshape(b, c, h, w)
    return q
